# baseline (speedup 1.0000x reference)
import functools
import jax
import jax.numpy as jnp
from jax import lax
from jax.experimental import pallas as pl
from jax.experimental.pallas import tpu as pltpu
from jax.experimental.pallas import tpu_sc as plsc

B = 32
VN = 16
PN = 1562
D = 64
XD = 16
NV = B * VN
NP = B * PN
EPG = PN * 16
EVG = VN * 4
RB = 6248
DCH = 2272
SCH = 176
SGB = 88

_MESH = plsc.VectorSubcoreMesh(core_axis_name="c", subcore_axis_name="s")
_f32 = jnp.float32


@functools.partial(
    pl.kernel,
    mesh=_MESH,
    compiler_params=pltpu.CompilerParams(use_tc_tiling_on_sc=False),
    out_type=[
        jax.ShapeDtypeStruct((B, PN, 16), _f32),
        jax.ShapeDtypeStruct((B, VN, 16), _f32),
    ],
    scratch_types=[
        pltpu.VMEM((PN, 16), _f32),
        pltpu.VMEM((DCH,), jnp.int32),
        pltpu.VMEM((VN, 16), _f32),
        pltpu.VMEM((EVG,), jnp.int32),
    ],
)
def _deg_kernel(pdst, vdst, zp, zv, outp, outv, strip, dbuf, stripv, dbufv):
    g = lax.axis_index("c") * 16 + lax.axis_index("s")
    ones = jnp.ones((16,), _f32)

    pltpu.sync_copy(zp, strip)
    nb = g * PN

    def pchunk(c, carry):
        pltpu.sync_copy(pdst.at[pl.ds(g * EPG + c * DCH, DCH)], dbuf)

        @plsc.parallel_loop(0, DCH // 16, unroll=4)
        def body(t):
            dv = dbuf[pl.ds(t * 16, 16)] - nb
            for l in range(16):
                plsc.addupdate(strip.at[dv[l]], ones)

        return carry

    lax.fori_loop(0, EPG // DCH, pchunk, 0)
    pltpu.sync_copy(strip, outp.at[g])

    pltpu.sync_copy(zv, stripv)
    pltpu.sync_copy(vdst.at[pl.ds(g * EVG, EVG)], dbufv)
    nbv = g * VN

    def vbody(t, cc):
        dv = dbufv[pl.ds(t * 16, 16)] - nbv
        for l in range(16):
            plsc.addupdate(stripv.at[dv[l]], ones)
        return cc

    lax.fori_loop(0, EVG // 16, vbody, 0)
    pltpu.sync_copy(stripv, outv.at[g])


def _make_spmm(fw, sch):
    nch = EPG // sch
    sgb = 88
    nsub = sch // sgb

    @functools.partial(
        pl.kernel,
        mesh=_MESH,
        compiler_params=pltpu.CompilerParams(use_tc_tiling_on_sc=False),
        out_type=[
            jax.ShapeDtypeStruct((B, PN * fw), _f32),
            jax.ShapeDtypeStruct((B, VN * fw), _f32),
        ],
        scratch_types=[
            pltpu.VMEM((PN * fw,), _f32),
            pltpu.VMEM((2, sch), jnp.int32),
            pltpu.VMEM((2, sch), jnp.int32),
            pltpu.VMEM((2, sch, fw), _f32),
            pltpu.VMEM((VN * fw,), _f32),
            pltpu.VMEM((EVG,), jnp.int32),
            pltpu.VMEM((EVG,), jnp.int32),
            pltpu.VMEM((EVG, fw), _f32),
            pltpu.SemaphoreType.DMA,
            pltpu.SemaphoreType.DMA,
        ],
    )
    def spmm(u_p, src_p, dst_p, u_v, src_v, dst_v, zp,
             outp, outv,
             acc, sidx, dbuf, rows, accv, sidxv, dbufv, rowsv,
             sem_i, sem_r):
        g = lax.axis_index("c") * 16 + lax.axis_index("s")
        nb = g * PN

        def issue_idx(c, s):
            cb = g * EPG + c * sch
            pltpu.async_copy(src_p.at[pl.ds(cb, sch)], sidx.at[s], sem_i)
            pltpu.async_copy(dst_p.at[pl.ds(cb, sch)], dbuf.at[s], sem_i)

        def wait_idx(s):
            pltpu.make_async_copy(src_p.at[pl.ds(0, sch)], sidx.at[s],
                                  sem_i).wait()
            pltpu.make_async_copy(dst_p.at[pl.ds(0, sch)], dbuf.at[s],
                                  sem_i).wait()

        def fire_gathers(s):
            for j in range(nsub):
                pltpu.async_copy(
                    u_p.at[sidx.at[s].at[pl.ds(j * sgb, sgb)]],
                    rows.at[s].at[pl.ds(j * sgb, sgb)], sem_r)

        def wait_gathers(s):
            for j in range(nsub):
                pltpu.make_async_copy(
                    u_p.at[sidx.at[s].at[pl.ds(j * sgb, sgb)]],
                    rows.at[s].at[pl.ds(j * sgb, sgb)], sem_r).wait()

        def accumulate(s):
            @plsc.parallel_loop(0, sch // 16, unroll=3)
            def body(t):
                offs = (dbuf[s, pl.ds(t * 16, 16)] - nb) * fw
                for l in range(16):
                    o = offs[l]
                    i = t * 16 + l
                    for k in range(fw // 16):
                        plsc.addupdate(acc.at[pl.ds(o + k * 16, 16)],
                                       rows[s, i, pl.ds(k * 16, 16)])

        pltpu.sync_copy(zp, acc)
        issue_idx(0, 0)
        issue_idx(1, 1)
        wait_idx(0)
        fire_gathers(0)

        def pair(t, cc):
            for s in (0, 1):
                c = t * 2 + s
                wait_gathers(s)

                @pl.when(c + 1 < nch)
                def _():
                    wait_idx(1 - s)
                    fire_gathers(1 - s)

                accumulate(s)

                @pl.when(c + 2 < nch)
                def _():
                    issue_idx(c + 2, s)

            return cc

        lax.fori_loop(0, nch // 2, pair, 0)
        if nch % 2 == 1:
            wait_gathers(0)
            accumulate(0)
        pltpu.sync_copy(acc, outp.at[g])

        pltpu.sync_copy(zp.at[pl.ds(0, VN * fw)], accv)
        pltpu.sync_copy(src_v.at[pl.ds(g * EVG, EVG)], sidxv)
        pltpu.sync_copy(dst_v.at[pl.ds(g * EVG, EVG)], dbufv)
        pltpu.async_copy(u_v.at[sidxv], rowsv, sem_r).wait()
        nbv = g * VN

        def vbody(t, cc):
            offs = (dbufv[pl.ds(t * 16, 16)] - nbv) * fw
            for l in range(16):
                o = offs[l]
                i = t * 16 + l
                for k in range(fw // 16):
                    plsc.addupdate(accv.at[pl.ds(o + k * 16, 16)],
                                   rowsv[i, pl.ds(k * 16, 16)])
            return cc

        lax.fori_loop(0, EVG // 16, vbody, 0)
        pltpu.sync_copy(accv, outv.at[g])

    return spmm


_spmm32 = _make_spmm(32, 352)
_spmm64 = _make_spmm(64, 176)


def _prep_body(x_ref, strip_ref, wi_ref, bi_ref, h0_ref, u0_ref, dinv_ref):
    x = x_ref[...]
    h0 = jnp.dot(x, wi_ref[...], preferred_element_type=_f32) + bi_ref[...]
    dinv = lax.rsqrt(strip_ref[:, 0:1] + 1.0)
    h0_ref[...] = h0
    u0_ref[...] = jnp.concatenate(
        [dinv * x, dinv, jnp.zeros_like(x)[:, :15]], axis=1)
    dinv_ref[...] = dinv


def _make_prep(n, rb):
    return pl.pallas_call(
        _prep_body,
        grid=(n // rb,),
        in_specs=[
            pl.BlockSpec((rb, XD), lambda i: (i, 0)),
            pl.BlockSpec((rb, 16), lambda i: (i, 0)),
            pl.BlockSpec((XD, D), lambda i: (0, 0)),
            pl.BlockSpec((1, D), lambda i: (0, 0)),
        ],
        out_specs=[
            pl.BlockSpec((rb, D), lambda i: (i, 0)),
            pl.BlockSpec((rb, 32), lambda i: (i, 0)),
            pl.BlockSpec((rb, 1), lambda i: (i, 0)),
        ],
        out_shape=[
            jax.ShapeDtypeStruct((n, D), _f32),
            jax.ShapeDtypeStruct((n, 32), _f32),
            jax.ShapeDtypeStruct((n, 1), _f32),
        ],
    )


def _mid_body(e0_ref, h0_ref, dinv_ref, w01_ref, w_ref, b_ref,
              h1_ref, u1_ref):
    dinv = dinv_ref[...]
    sh = dinv * jnp.dot(e0_ref[...], w01_ref[...],
                        preferred_element_type=_f32) \
        + dinv * dinv * h0_ref[...]
    h1 = jnp.maximum(
        jnp.dot(sh, w_ref[...], preferred_element_type=_f32) + b_ref[...], 0.0)
    h1_ref[...] = h1
    u1_ref[...] = dinv * h1


def _make_mid(n, rb):
    return pl.pallas_call(
        _mid_body,
        grid=(n // rb,),
        in_specs=[
            pl.BlockSpec((rb, 32), lambda i: (i, 0)),
            pl.BlockSpec((rb, D), lambda i: (i, 0)),
            pl.BlockSpec((rb, 1), lambda i: (i, 0)),
            pl.BlockSpec((32, D), lambda i: (0, 0)),
            pl.BlockSpec((D, D), lambda i: (0, 0)),
            pl.BlockSpec((1, D), lambda i: (0, 0)),
        ],
        out_specs=[
            pl.BlockSpec((rb, D), lambda i: (i, 0)),
            pl.BlockSpec((rb, D), lambda i: (i, 0)),
        ],
        out_shape=[
            jax.ShapeDtypeStruct((n, D), _f32),
            jax.ShapeDtypeStruct((n, D), _f32),
        ],
    )


def _vadd_body(e1_ref, h1_ref, h0_ref, dinv_ref, w_ref, b_ref, ids_ref,
               out_ref):
    dinv = dinv_ref[...]
    sh = dinv * e1_ref[...] + dinv * dinv * h1_ref[...]
    h2 = jnp.dot(sh, w_ref[...], preferred_element_type=_f32) + b_ref[...]
    gid = lax.broadcasted_iota(jnp.int32, (B, NV), 0)
    nid = lax.broadcasted_iota(jnp.int32, (B, NV), 1)
    pool = jnp.where(nid // VN == gid, 1.0 / VN, 0.0)
    m = jnp.dot(pool, h2, preferred_element_type=_f32)
    tgt = ids_ref[...] + VN * lax.broadcasted_iota(jnp.int32, (B, 1), 0)
    sel = jnp.where(nid == tgt, 1.0, 0.0)
    rows = jnp.dot(sel, h2 + h0_ref[...], preferred_element_type=_f32)
    out_ref[...] = 2.0 * m + rows


_vadd_call = pl.pallas_call(
    _vadd_body,
    grid=(1,),
    in_specs=[
        pl.BlockSpec((NV, D), lambda i: (0, 0)),
        pl.BlockSpec((NV, D), lambda i: (0, 0)),
        pl.BlockSpec((NV, D), lambda i: (0, 0)),
        pl.BlockSpec((NV, 1), lambda i: (0, 0)),
        pl.BlockSpec((D, D), lambda i: (0, 0)),
        pl.BlockSpec((1, D), lambda i: (0, 0)),
        pl.BlockSpec((B, 1), lambda i: (0, 0)),
    ],
    out_specs=pl.BlockSpec((B, D), lambda i: (0, 0)),
    out_shape=jax.ShapeDtypeStruct((B, D), _f32),
)


def _final_body(e1_ref, h1_ref, h0_ref, dinv_ref, w2_ref, b2_ref, vadd_ref,
                aw1_ref, ab1_ref, aw2_ref, ab2_ref, out_ref):
    dinv = dinv_ref[0]
    sh = dinv * e1_ref[0] + dinv * dinv * h1_ref[0]
    h2 = jnp.dot(sh, w2_ref[...], preferred_element_type=_f32) + b2_ref[...]
    gemb = jnp.mean(h2, axis=0, keepdims=True)
    state = h2 + gemb + h0_ref[0] + vadd_ref[0]
    t = jnp.maximum(
        jnp.dot(state, aw1_ref[...], preferred_element_type=_f32)
        + ab1_ref[...], 0.0)
    s = jnp.sum(t * aw2_ref[...], axis=1)
    out_ref[0] = s[None, :] + ab2_ref[...]


_final_call = pl.pallas_call(
    _final_body,
    grid=(B,),
    in_specs=[
        pl.BlockSpec((1, PN, D), lambda g: (g, 0, 0)),
        pl.BlockSpec((1, PN, D), lambda g: (g, 0, 0)),
        pl.BlockSpec((1, PN, D), lambda g: (g, 0, 0)),
        pl.BlockSpec((1, PN, 1), lambda g: (g, 0, 0)),
        pl.BlockSpec((D, D), lambda g: (0, 0)),
        pl.BlockSpec((1, D), lambda g: (0, 0)),
        pl.BlockSpec((1, 1, D), lambda g: (g, 0, 0)),
        pl.BlockSpec((D, D), lambda g: (0, 0)),
        pl.BlockSpec((1, D), lambda g: (0, 0)),
        pl.BlockSpec((1, D), lambda g: (0, 0)),
        pl.BlockSpec((1, 1), lambda g: (0, 0)),
    ],
    out_specs=pl.BlockSpec((1, 1, PN), lambda g: (g, 0, 0)),
    out_shape=jax.ShapeDtypeStruct((B, 1, PN), _f32),
)


def kernel(v_x, p_x, v_edge_index, p_edge_index, curr_v_node_id,
           v_init_W, v_init_b, v_gnn_W1, v_gnn_b1, v_gnn_W2, v_gnn_b2,
           p_init_W, p_init_b, p_gnn_W1, p_gnn_b1, p_gnn_W2, p_gnn_b2,
           actor_W1, actor_b1, actor_W2, actor_b2):
    src_p, dst_p = p_edge_index[0], p_edge_index[1]
    src_v, dst_v = v_edge_index[0], v_edge_index[1]
    zp16 = jnp.zeros((PN, 16), _f32)
    zv16 = jnp.zeros((VN, 16), _f32)
    zp64 = jnp.zeros((PN * D,), _f32)
    zp32 = jnp.zeros((PN * 32,), _f32)
    w01p = jnp.concatenate(
        [p_init_W, p_init_b.reshape(1, D),
         jnp.zeros((15, D), _f32)], axis=0)
    w01v = jnp.concatenate(
        [v_init_W, v_init_b.reshape(1, D),
         jnp.zeros((15, D), _f32)], axis=0)

    degp, degv = _deg_kernel(dst_p, dst_v, zp16, zv16)
    degp = degp.reshape(NP, 16)
    degv = degv.reshape(NV, 16)

    h0p, u0p, dinvp = _make_prep(NP, RB)(p_x, degp, p_init_W,
                                          p_init_b.reshape(1, D))
    h0v, u0v, dinvv = _make_prep(NV, NV)(v_x, degv, v_init_W,
                                         v_init_b.reshape(1, D))

    e0p, e0v = _spmm32(u0p, src_p, dst_p, u0v, src_v, dst_v, zp32)
    e0p = e0p.reshape(NP, 32)
    e0v = e0v.reshape(NV, 32)

    h1p, u1p = _make_mid(NP, RB)(e0p, h0p, dinvp, w01p, p_gnn_W1,
                                 p_gnn_b1.reshape(1, D))
    h1v, u1v = _make_mid(NV, NV)(e0v, h0v, dinvv, w01v, v_gnn_W1,
                                 v_gnn_b1.reshape(1, D))

    e1p, e1v = _spmm64(u1p, src_p, dst_p, u1v, src_v, dst_v, zp64)
    e1v = e1v.reshape(NV, D)
    e1p = e1p.reshape(B, PN, D)

    vadd = _vadd_call(e1v, h1v, h0v, dinvv, v_gnn_W2,
                      v_gnn_b2.reshape(1, D),
                      curr_v_node_id.reshape(B, 1).astype(jnp.int32))

    logits = _final_call(
        e1p, h1p.reshape(B, PN, D), h0p.reshape(B, PN, D),
        dinvp.reshape(B, PN, 1), p_gnn_W2, p_gnn_b2.reshape(1, D),
        vadd.reshape(B, 1, D), actor_W1, actor_b1.reshape(1, D),
        actor_W2.reshape(1, D), actor_b2.reshape(1, 1))
    return logits.reshape(B, PN)


# speedup vs baseline: 1.1132x; 1.1132x over previous
import functools
import jax
import jax.numpy as jnp
from jax import lax
from jax.experimental import pallas as pl
from jax.experimental.pallas import tpu as pltpu
from jax.experimental.pallas import tpu_sc as plsc

B = 32
VN = 16
PN = 1562
D = 64
XD = 16
NV = B * VN
NP = B * PN
EPG = PN * 16
EVG = VN * 4
RB = 6248
DCH = 2272
SCH = 176
SGB = 88

_MESH = plsc.VectorSubcoreMesh(core_axis_name="c", subcore_axis_name="s")
_f32 = jnp.float32


@functools.partial(
    pl.kernel,
    mesh=_MESH,
    compiler_params=pltpu.CompilerParams(use_tc_tiling_on_sc=False),
    out_type=[
        jax.ShapeDtypeStruct((B, PN, 16), _f32),
        jax.ShapeDtypeStruct((B, VN, 16), _f32),
    ],
    scratch_types=[
        pltpu.VMEM((PN, 16), _f32),
        pltpu.VMEM((DCH,), jnp.int32),
        pltpu.VMEM((VN, 16), _f32),
        pltpu.VMEM((EVG,), jnp.int32),
    ],
)
def _deg_kernel(pdst, vdst, zp, zv, outp, outv, strip, dbuf, stripv, dbufv):
    g = lax.axis_index("c") * 16 + lax.axis_index("s")
    ones = jnp.ones((16,), _f32)

    pltpu.sync_copy(zp, strip)
    nb = g * PN

    def pchunk(c, carry):
        pltpu.sync_copy(pdst.at[pl.ds(g * EPG + c * DCH, DCH)], dbuf)

        @plsc.parallel_loop(0, DCH // 16, unroll=4)
        def body(t):
            dv = dbuf[pl.ds(t * 16, 16)] - nb
            for l in range(16):
                plsc.addupdate(strip.at[dv[l]], ones)

        return carry

    lax.fori_loop(0, EPG // DCH, pchunk, 0)
    pltpu.sync_copy(strip, outp.at[g])

    pltpu.sync_copy(zv, stripv)
    pltpu.sync_copy(vdst.at[pl.ds(g * EVG, EVG)], dbufv)
    nbv = g * VN

    def vbody(t, cc):
        dv = dbufv[pl.ds(t * 16, 16)] - nbv
        for l in range(16):
            plsc.addupdate(stripv.at[dv[l]], ones)
        return cc

    lax.fori_loop(0, EVG // 16, vbody, 0)
    pltpu.sync_copy(stripv, outv.at[g])


def _make_spmm(fw, sch):
    nch = EPG // sch
    sgb = 88
    nsub = sch // sgb

    @functools.partial(
        pl.kernel,
        mesh=_MESH,
        compiler_params=pltpu.CompilerParams(use_tc_tiling_on_sc=False),
        out_type=[
            jax.ShapeDtypeStruct((B, PN * fw), _f32),
            jax.ShapeDtypeStruct((B, VN * fw), _f32),
        ],
        scratch_types=[
            pltpu.VMEM((PN * fw,), _f32),
            pltpu.VMEM((2, sch), jnp.int32),
            pltpu.VMEM((2, sch), jnp.int32),
            pltpu.VMEM((2, sch, fw), _f32),
            pltpu.VMEM((VN * fw,), _f32),
            pltpu.VMEM((EVG,), jnp.int32),
            pltpu.VMEM((EVG,), jnp.int32),
            pltpu.VMEM((EVG, fw), _f32),
            pltpu.SemaphoreType.DMA,
            pltpu.SemaphoreType.DMA,
        ],
    )
    def spmm(u_p, src_p, dst_p, u_v, src_v, dst_v, zp,
             outp, outv,
             acc, sidx, dbuf, rows, accv, sidxv, dbufv, rowsv,
             sem_i, sem_r):
        g = lax.axis_index("c") * 16 + lax.axis_index("s")
        nb = g * PN

        def issue_idx(c, s):
            cb = g * EPG + c * sch
            pltpu.async_copy(src_p.at[pl.ds(cb, sch)], sidx.at[s], sem_i)
            pltpu.async_copy(dst_p.at[pl.ds(cb, sch)], dbuf.at[s], sem_i)

        def wait_idx(s):
            pltpu.make_async_copy(src_p.at[pl.ds(0, sch)], sidx.at[s],
                                  sem_i).wait()
            pltpu.make_async_copy(dst_p.at[pl.ds(0, sch)], dbuf.at[s],
                                  sem_i).wait()

        def fire_gathers(s):
            for j in range(nsub):
                pltpu.async_copy(
                    u_p.at[sidx.at[s].at[pl.ds(j * sgb, sgb)]],
                    rows.at[s].at[pl.ds(j * sgb, sgb)], sem_r)

        def wait_gathers(s):
            for j in range(nsub):
                pltpu.make_async_copy(
                    u_p.at[sidx.at[s].at[pl.ds(j * sgb, sgb)]],
                    rows.at[s].at[pl.ds(j * sgb, sgb)], sem_r).wait()

        def accumulate(s):
            @plsc.parallel_loop(0, sch // 16, unroll=2)
            def body(t):
                offs = (dbuf[s, pl.ds(t * 16, 16)] - nb) * fw
                for l in range(16):
                    o = offs[l]
                    i = t * 16 + l
                    for k in range(fw // 16):
                        plsc.addupdate(acc.at[pl.ds(o + k * 16, 16)],
                                       rows[s, i, pl.ds(k * 16, 16)])

        pltpu.sync_copy(zp, acc)
        issue_idx(0, 0)
        issue_idx(1, 1)
        wait_idx(0)
        fire_gathers(0)

        def pair(t, cc):
            for s in (0, 1):
                c = t * 2 + s
                wait_gathers(s)

                @pl.when(c + 1 < nch)
                def _():
                    wait_idx(1 - s)
                    fire_gathers(1 - s)

                accumulate(s)

                @pl.when(c + 2 < nch)
                def _():
                    issue_idx(c + 2, s)

            return cc

        lax.fori_loop(0, nch // 2, pair, 0)
        if nch % 2 == 1:
            wait_gathers(0)
            accumulate(0)
        pltpu.sync_copy(acc, outp.at[g])

        pltpu.sync_copy(zp.at[pl.ds(0, VN * fw)], accv)
        pltpu.sync_copy(src_v.at[pl.ds(g * EVG, EVG)], sidxv)
        pltpu.sync_copy(dst_v.at[pl.ds(g * EVG, EVG)], dbufv)
        pltpu.async_copy(u_v.at[sidxv], rowsv, sem_r).wait()
        nbv = g * VN

        def vbody(t, cc):
            offs = (dbufv[pl.ds(t * 16, 16)] - nbv) * fw
            for l in range(16):
                o = offs[l]
                i = t * 16 + l
                for k in range(fw // 16):
                    plsc.addupdate(accv.at[pl.ds(o + k * 16, 16)],
                                   rowsv[i, pl.ds(k * 16, 16)])
            return cc

        lax.fori_loop(0, EVG // 16, vbody, 0)
        pltpu.sync_copy(accv, outv.at[g])

    return spmm


_spmm32 = _make_spmm(32, 352)
_spmm64 = _make_spmm(64, 176)


def _prep_body(x_ref, strip_ref, wi_ref, bi_ref, h0_ref, u0_ref, dinv_ref):
    x = x_ref[...]
    h0 = jnp.dot(x, wi_ref[...], preferred_element_type=_f32) + bi_ref[...]
    dinv = lax.rsqrt(strip_ref[:, 0:1] + 1.0)
    h0_ref[...] = h0
    u0_ref[...] = jnp.concatenate(
        [dinv * x, dinv, jnp.zeros_like(x)[:, :15]], axis=1)
    dinv_ref[...] = dinv


def _make_prep(n, rb):
    return pl.pallas_call(
        _prep_body,
        grid=(n // rb,),
        in_specs=[
            pl.BlockSpec((rb, XD), lambda i: (i, 0)),
            pl.BlockSpec((rb, 16), lambda i: (i, 0)),
            pl.BlockSpec((XD, D), lambda i: (0, 0)),
            pl.BlockSpec((1, D), lambda i: (0, 0)),
        ],
        out_specs=[
            pl.BlockSpec((rb, D), lambda i: (i, 0)),
            pl.BlockSpec((rb, 32), lambda i: (i, 0)),
            pl.BlockSpec((rb, 1), lambda i: (i, 0)),
        ],
        out_shape=[
            jax.ShapeDtypeStruct((n, D), _f32),
            jax.ShapeDtypeStruct((n, 32), _f32),
            jax.ShapeDtypeStruct((n, 1), _f32),
        ],
    )


def _mid_body(e0_ref, h0_ref, dinv_ref, w01_ref, w_ref, b_ref,
              h1_ref, u1_ref):
    dinv = dinv_ref[...]
    sh = dinv * jnp.dot(e0_ref[...], w01_ref[...],
                        preferred_element_type=_f32) \
        + dinv * dinv * h0_ref[...]
    h1 = jnp.maximum(
        jnp.dot(sh, w_ref[...], preferred_element_type=_f32) + b_ref[...], 0.0)
    h1_ref[...] = h1
    u1_ref[...] = dinv * h1


def _make_mid(n, rb):
    return pl.pallas_call(
        _mid_body,
        grid=(n // rb,),
        in_specs=[
            pl.BlockSpec((rb, 32), lambda i: (i, 0)),
            pl.BlockSpec((rb, D), lambda i: (i, 0)),
            pl.BlockSpec((rb, 1), lambda i: (i, 0)),
            pl.BlockSpec((32, D), lambda i: (0, 0)),
            pl.BlockSpec((D, D), lambda i: (0, 0)),
            pl.BlockSpec((1, D), lambda i: (0, 0)),
        ],
        out_specs=[
            pl.BlockSpec((rb, D), lambda i: (i, 0)),
            pl.BlockSpec((rb, D), lambda i: (i, 0)),
        ],
        out_shape=[
            jax.ShapeDtypeStruct((n, D), _f32),
            jax.ShapeDtypeStruct((n, D), _f32),
        ],
    )


def _vadd_body(e1_ref, h1_ref, h0_ref, dinv_ref, w_ref, b_ref, ids_ref,
               out_ref):
    dinv = dinv_ref[...]
    sh = dinv * e1_ref[...] + dinv * dinv * h1_ref[...]
    h2 = jnp.dot(sh, w_ref[...], preferred_element_type=_f32) + b_ref[...]
    gid = lax.broadcasted_iota(jnp.int32, (B, NV), 0)
    nid = lax.broadcasted_iota(jnp.int32, (B, NV), 1)
    pool = jnp.where(nid // VN == gid, 1.0 / VN, 0.0)
    m = jnp.dot(pool, h2, preferred_element_type=_f32)
    tgt = ids_ref[...] + VN * lax.broadcasted_iota(jnp.int32, (B, 1), 0)
    sel = jnp.where(nid == tgt, 1.0, 0.0)
    rows = jnp.dot(sel, h2 + h0_ref[...], preferred_element_type=_f32)
    out_ref[...] = 2.0 * m + rows


_vadd_call = pl.pallas_call(
    _vadd_body,
    grid=(1,),
    in_specs=[
        pl.BlockSpec((NV, D), lambda i: (0, 0)),
        pl.BlockSpec((NV, D), lambda i: (0, 0)),
        pl.BlockSpec((NV, D), lambda i: (0, 0)),
        pl.BlockSpec((NV, 1), lambda i: (0, 0)),
        pl.BlockSpec((D, D), lambda i: (0, 0)),
        pl.BlockSpec((1, D), lambda i: (0, 0)),
        pl.BlockSpec((B, 1), lambda i: (0, 0)),
    ],
    out_specs=pl.BlockSpec((B, D), lambda i: (0, 0)),
    out_shape=jax.ShapeDtypeStruct((B, D), _f32),
)


def _final_body(e1_ref, h1_ref, h0_ref, dinv_ref, w2_ref, b2_ref, vadd_ref,
                aw1_ref, ab1_ref, aw2_ref, ab2_ref, out_ref):
    dinv = dinv_ref[0]
    sh = dinv * e1_ref[0] + dinv * dinv * h1_ref[0]
    h2 = jnp.dot(sh, w2_ref[...], preferred_element_type=_f32) + b2_ref[...]
    gemb = jnp.mean(h2, axis=0, keepdims=True)
    state = h2 + gemb + h0_ref[0] + vadd_ref[0]
    t = jnp.maximum(
        jnp.dot(state, aw1_ref[...], preferred_element_type=_f32)
        + ab1_ref[...], 0.0)
    s = jnp.sum(t * aw2_ref[...], axis=1)
    out_ref[0] = s[None, :] + ab2_ref[...]


_final_call = pl.pallas_call(
    _final_body,
    grid=(B,),
    in_specs=[
        pl.BlockSpec((1, PN, D), lambda g: (g, 0, 0)),
        pl.BlockSpec((1, PN, D), lambda g: (g, 0, 0)),
        pl.BlockSpec((1, PN, D), lambda g: (g, 0, 0)),
        pl.BlockSpec((1, PN, 1), lambda g: (g, 0, 0)),
        pl.BlockSpec((D, D), lambda g: (0, 0)),
        pl.BlockSpec((1, D), lambda g: (0, 0)),
        pl.BlockSpec((1, 1, D), lambda g: (g, 0, 0)),
        pl.BlockSpec((D, D), lambda g: (0, 0)),
        pl.BlockSpec((1, D), lambda g: (0, 0)),
        pl.BlockSpec((1, D), lambda g: (0, 0)),
        pl.BlockSpec((1, 1), lambda g: (0, 0)),
    ],
    out_specs=pl.BlockSpec((1, 1, PN), lambda g: (g, 0, 0)),
    out_shape=jax.ShapeDtypeStruct((B, 1, PN), _f32),
)


def kernel(v_x, p_x, v_edge_index, p_edge_index, curr_v_node_id,
           v_init_W, v_init_b, v_gnn_W1, v_gnn_b1, v_gnn_W2, v_gnn_b2,
           p_init_W, p_init_b, p_gnn_W1, p_gnn_b1, p_gnn_W2, p_gnn_b2,
           actor_W1, actor_b1, actor_W2, actor_b2):
    src_p, dst_p = p_edge_index[0], p_edge_index[1]
    src_v, dst_v = v_edge_index[0], v_edge_index[1]
    zp16 = jnp.zeros((PN, 16), _f32)
    zv16 = jnp.zeros((VN, 16), _f32)
    zp64 = jnp.zeros((PN * D,), _f32)
    zp32 = jnp.zeros((PN * 32,), _f32)
    w01p = jnp.concatenate(
        [p_init_W, p_init_b.reshape(1, D),
         jnp.zeros((15, D), _f32)], axis=0)
    w01v = jnp.concatenate(
        [v_init_W, v_init_b.reshape(1, D),
         jnp.zeros((15, D), _f32)], axis=0)

    degp, degv = _deg_kernel(dst_p, dst_v, zp16, zv16)
    degp = degp.reshape(NP, 16)
    degv = degv.reshape(NV, 16)

    h0p, u0p, dinvp = _make_prep(NP, RB)(p_x, degp, p_init_W,
                                          p_init_b.reshape(1, D))
    h0v, u0v, dinvv = _make_prep(NV, NV)(v_x, degv, v_init_W,
                                         v_init_b.reshape(1, D))

    e0p, e0v = _spmm32(u0p, src_p, dst_p, u0v, src_v, dst_v, zp32)
    e0p = e0p.reshape(NP, 32)
    e0v = e0v.reshape(NV, 32)

    h1p, u1p = _make_mid(NP, RB)(e0p, h0p, dinvp, w01p, p_gnn_W1,
                                 p_gnn_b1.reshape(1, D))
    h1v, u1v = _make_mid(NV, NV)(e0v, h0v, dinvv, w01v, v_gnn_W1,
                                 v_gnn_b1.reshape(1, D))

    e1p, e1v = _spmm64(u1p, src_p, dst_p, u1v, src_v, dst_v, zp64)
    e1v = e1v.reshape(NV, D)
    e1p = e1p.reshape(B, PN, D)

    vadd = _vadd_call(e1v, h1v, h0v, dinvv, v_gnn_W2,
                      v_gnn_b2.reshape(1, D),
                      curr_v_node_id.reshape(B, 1).astype(jnp.int32))

    logits = _final_call(
        e1p, h1p.reshape(B, PN, D), h0p.reshape(B, PN, D),
        dinvp.reshape(B, PN, 1), p_gnn_W2, p_gnn_b2.reshape(1, D),
        vadd.reshape(B, 1, D), actor_W1, actor_b1.reshape(1, D),
        actor_W2.reshape(1, D), actor_b2.reshape(1, 1))
    return logits.reshape(B, PN)
